# Initial kernel scaffold; baseline (speedup 1.0000x reference)
import jax
import jax.numpy as jnp
from jax.experimental import pallas as pl


def kernel(xyz, points, w0, b0, g0, be0, w1, b1, g1, be1, w2, b2, g2, be2):
    raise NotImplementedError("write your pallas kernel here")


# speedup vs baseline: 15.6701x; 15.6701x over previous
import functools

import jax
import jax.numpy as jnp
from jax import lax
from jax.experimental import pallas as pl
from jax.experimental.pallas import tpu as pltpu
from jax.experimental.pallas import tpu_sc as plsc

B = 8
N = 4096
NPOINT = 1024
RADIUS = 0.4
NSAMPLE = 32
EPS = 1e-5
R2 = RADIUS * RADIUS
M_TOT = B * NPOINT * NSAMPLE


def _fps_body(xyz_ref, out_ref):
    X = xyz_ref[:, 0, :]
    Y = xyz_ref[:, 1, :]
    Z = xyz_ref[:, 2, :]
    iota = lax.broadcasted_iota(jnp.int32, (B, N), 1)

    def body(i, carry):
        distance, farthest = carry
        oh = (iota == farthest).astype(jnp.float32)
        cx = jnp.sum(X * oh, axis=1, keepdims=True)
        cy = jnp.sum(Y * oh, axis=1, keepdims=True)
        cz = jnp.sum(Z * oh, axis=1, keepdims=True)
        cxyz = jnp.concatenate([cx, cy, cz], axis=1)
        out_ref[pl.ds(i, 1)] = cxyz.reshape(1, B, 3)
        dx = X - cx
        dy = Y - cy
        dz = Z - cz
        d = (dx * dx + dy * dy) + dz * dz
        distance = jnp.minimum(distance, d)
        mx = jnp.max(distance, axis=1, keepdims=True)
        elig = distance == mx
        farthest = jnp.min(jnp.where(elig, iota, N), axis=1, keepdims=True)
        return distance, farthest

    init = (jnp.full((B, N), 1e10, dtype=jnp.float32),
            jnp.zeros((B, 1), dtype=jnp.int32))
    lax.fori_loop(0, NPOINT, body, init)


def _fps(xyz):
    out = pl.pallas_call(
        _fps_body,
        out_shape=jax.ShapeDtypeStruct((NPOINT, B, 3), jnp.float32),
    )(xyz)
    return jnp.transpose(out, (1, 2, 0))


S_TILE = 128
DN = (((0,), (0,)), ((), ()))


def _ballq_body(xyz_ref, nxyzt_ref, srcn_ref, dstn_ref, out_ref):
    b = pl.program_id(0)
    nxT = nxyzt_ref[0]
    X3 = xyz_ref[0]
    dot = lax.dot_general(nxT.astype(jnp.bfloat16), X3.astype(jnp.bfloat16),
                          (((1,), (0,)), ((), ())),
                          preferred_element_type=jnp.float32)
    srcn = srcn_ref[0]
    dstn = dstn_ref[0]
    sq = -2.0 * dot + srcn + dstn
    m = (sq <= R2).astype(jnp.float32)
    tri = (lax.broadcasted_iota(jnp.int32, (128, 128), 0)
           <= lax.broadcasted_iota(jnp.int32, (128, 128), 1)).astype(jnp.float32)
    parts = []
    carry = jnp.zeros((S_TILE, 1), jnp.float32)
    for cblk in range(N // 128):
        ch = m[:, cblk * 128:(cblk + 1) * 128]
        cs = lax.dot_general(ch, tri, (((1,), (0,)), ((), ())),
                             preferred_element_type=jnp.float32) + carry
        parts.append(cs)
        carry = cs[:, 127:128]
    pos = jnp.concatenate(parts, axis=1)
    count = pos[:, N - 1:N]
    iota = lax.broadcasted_iota(jnp.int32, (S_TILE, N), 1).astype(jnp.float32)
    cols = []
    first = None
    for j in range(NSAMPLE):
        ohj = m * (pos == jnp.float32(j + 1))
        vj = jnp.sum(iota * ohj, axis=1, keepdims=True)
        if j == 0:
            first = vj
        else:
            vj = jnp.where(count >= jnp.float32(j + 1), vj, first)
        cols.append(vj)
    idxm = jnp.concatenate(cols, axis=1)
    idxm = jnp.where(count >= 1.0, idxm, jnp.float32(N - 1))
    out_ref[0] = idxm.astype(jnp.int32).T + b * N


def _ball_query(xyz, new_xyz):
    new_xyz_t = jnp.transpose(new_xyz, (0, 2, 1))
    srcn = jnp.sum(new_xyz_t ** 2, -1)[:, :, None]
    dstn = jnp.sum(jnp.transpose(xyz, (0, 2, 1)) ** 2, -1)[:, None, :]
    return pl.pallas_call(
        _ballq_body,
        grid=(B, NPOINT // S_TILE),
        in_specs=[
            pl.BlockSpec((1, 3, N), lambda b, s: (b, 0, 0)),
            pl.BlockSpec((1, S_TILE, 3), lambda b, s: (b, s, 0)),
            pl.BlockSpec((1, S_TILE, 1), lambda b, s: (b, s, 0)),
            pl.BlockSpec((1, 1, N), lambda b, s: (b, 0, 0)),
        ],
        out_specs=pl.BlockSpec((1, NSAMPLE, S_TILE), lambda b, s: (b, 0, s)),
        out_shape=jax.ShapeDtypeStruct((B, NSAMPLE, NPOINT), jnp.int32),
    )(xyz, new_xyz_t, srcn, dstn)


GCH = 512


def _sc_gather(table, idx):
    info = plsc.get_sparse_core_info()
    nc, ns = info.num_cores, info.num_subcores
    nw = nc * ns
    b_per_w = M_TOT // nw
    mesh = plsc.VectorSubcoreMesh(core_axis_name="c", subcore_axis_name="s")

    @functools.partial(
        pl.kernel, mesh=mesh,
        out_type=jax.ShapeDtypeStruct((M_TOT, 128), jnp.float32),
        scratch_types=[
            pltpu.VMEM((GCH,), jnp.int32),
            pltpu.VMEM((GCH, 128), jnp.float32),
            pltpu.SemaphoreType.DMA,
        ],
    )
    def k(table_hbm, idx_hbm, out_hbm, idx_v, rows_v, sem):
        wid = lax.axis_index("s") * nc + lax.axis_index("c")
        base = wid * b_per_w
        for ci in range(b_per_w // GCH):
            o = base + ci * GCH
            pltpu.sync_copy(idx_hbm.at[pl.ds(o, GCH)], idx_v)
            pltpu.async_copy(table_hbm.at[idx_v], rows_v, sem).wait()
            pltpu.sync_copy(rows_v, out_hbm.at[pl.ds(o, GCH)])

    return k(table, idx)


D_TILE = 4096


def _x0_body(g_ref, c_ref, x0_ref, mom_ref):
    first = pl.program_id(0) == 0

    @pl.when(first)
    def _():
        mom_ref[...] = jnp.zeros((16, 8), jnp.float32)

    g = g_ref[...]
    c = c_ref[...]
    x0 = jnp.concatenate(
        [g[:, 0:3] - c, g[:, 3:6], jnp.zeros((D_TILE, 2), jnp.float32)], axis=1)
    x0_ref[...] = x0
    sig = lax.dot_general(x0, x0, DN, preferred_element_type=jnp.float32)
    s = jnp.sum(x0, axis=0, keepdims=True)
    mom_ref[0:8, :] += sig
    mom_ref[8:9, :] += s


def _x0_build(g, centers):
    return pl.pallas_call(
        _x0_body,
        grid=(M_TOT // D_TILE,),
        in_specs=[
            pl.BlockSpec((D_TILE, 128), lambda i: (i, 0)),
            pl.BlockSpec((D_TILE, 3), lambda i: (i, 0)),
        ],
        out_specs=[
            pl.BlockSpec((D_TILE, 8), lambda i: (i, 0)),
            pl.BlockSpec((16, 8), lambda i: (0, 0)),
        ],
        out_shape=[
            jax.ShapeDtypeStruct((M_TOT, 8), jnp.float32),
            jax.ShapeDtypeStruct((16, 8), jnp.float32),
        ],
    )(g, centers)


L_TILE = 8192


def _layer_body(x_ref, w_ref, b_ref, y_ref, mom_ref, *, c_out):
    first = pl.program_id(0) == 0

    @pl.when(first)
    def _():
        mom_ref[...] = jnp.zeros((c_out + 8, c_out), jnp.float32)

    x = x_ref[...]
    y = lax.dot_general(x, w_ref[...], (((1,), (0,)), ((), ())),
                        preferred_element_type=jnp.float32) + b_ref[...]
    y = jnp.maximum(y, 0.0)
    y_ref[...] = y
    sig = lax.dot_general(y, y, DN, preferred_element_type=jnp.float32)
    s = jnp.sum(y, axis=0, keepdims=True)
    mom_ref[0:c_out, :] += sig
    mom_ref[c_out:c_out + 1, :] += s


def _layer(x, wp, bp, c_out):
    c_in = x.shape[1]
    return pl.pallas_call(
        functools.partial(_layer_body, c_out=c_out),
        grid=(M_TOT // L_TILE,),
        in_specs=[
            pl.BlockSpec((L_TILE, c_in), lambda i: (i, 0)),
            pl.BlockSpec((c_in, c_out), lambda i: (0, 0)),
            pl.BlockSpec((1, c_out), lambda i: (0, 0)),
        ],
        out_specs=[
            pl.BlockSpec((L_TILE, c_out), lambda i: (i, 0)),
            pl.BlockSpec((c_out + 8, c_out), lambda i: (0, 0)),
        ],
        out_shape=[
            jax.ShapeDtypeStruct((M_TOT, c_out), jnp.float32),
            jax.ShapeDtypeStruct((c_out + 8, c_out), jnp.float32),
        ],
    )(x, wp, bp)


G_TILE = 512


def _final_body(x_ref, w_ref, b_ref, out_ref):
    acc = jnp.full((G_TILE, 64), -jnp.inf, jnp.float32)
    for k in range(NSAMPLE):
        xk = x_ref[0, k]
        y = lax.dot_general(xk, w_ref[...], (((1,), (0,)), ((), ())),
                            preferred_element_type=jnp.float32) + b_ref[...]
        acc = jnp.maximum(acc, jnp.maximum(y, 0.0))
    out_ref[0] = acc


def _final(x2, wp, bp):
    x4 = x2.reshape(B, NSAMPLE, NPOINT, 32)
    return pl.pallas_call(
        _final_body,
        grid=(B, NPOINT // G_TILE),
        in_specs=[
            pl.BlockSpec((1, NSAMPLE, G_TILE, 32), lambda b, s: (b, 0, s, 0)),
            pl.BlockSpec((32, 64), lambda b, s: (0, 0)),
            pl.BlockSpec((1, 64), lambda b, s: (0, 0)),
        ],
        out_specs=pl.BlockSpec((1, G_TILE, 64), lambda b, s: (b, s, 0)),
        out_shape=jax.ShapeDtypeStruct((B, NPOINT, 64), jnp.float32),
    )(x4, wp, bp)


def _affine(w, b, g, be, mom, c_in, srow):
    sig = mom[0:c_in, 0:c_in]
    s = mom[srow, 0:c_in]
    m = jnp.float32(M_TOT)
    ws = w @ s
    mean = (ws + b * m) / m
    ey2 = (jnp.einsum('oc,cd,od->o', w, sig, w) + 2.0 * b * ws + m * b * b) / m
    var = ey2 - mean * mean
    scale = g / jnp.sqrt(var + EPS)
    wp = w * scale[:, None]
    bp = scale * (b - mean) + be
    return wp, bp


def _pack(wp, bp, c_in_pad, c_out):
    wt = jnp.zeros((c_in_pad, c_out), jnp.float32).at[0:wp.shape[1]].set(wp.T)
    return wt, bp.reshape(1, c_out)


def kernel(xyz, points, w0, b0, g0, be0, w1, b1, g1, be1, w2, b2, g2, be2):
    new_xyz = _fps(xyz)
    idx = _ball_query(xyz, new_xyz)

    table = jnp.concatenate([xyz, points], axis=1)
    table = jnp.transpose(table, (0, 2, 1)).reshape(B * N, 6)
    table = jnp.pad(table, ((0, 0), (0, 122)))
    flat_idx = idx.reshape(M_TOT)
    g = _sc_gather(table, flat_idx)

    centers = jnp.broadcast_to(
        jnp.transpose(new_xyz, (0, 2, 1))[:, None, :, :],
        (B, NSAMPLE, NPOINT, 3)).reshape(M_TOT, 3)
    x0, mom0 = _x0_build(g, centers)

    wp0, bp0 = _affine(w0, b0, g0, be0, mom0, 6, 8)
    w0t, b0t = _pack(wp0, bp0, 8, 32)
    x1, mom1 = _layer(x0, w0t, b0t, 32)

    wp1, bp1 = _affine(w1, b1, g1, be1, mom1, 32, 32)
    w1t, b1t = _pack(wp1, bp1, 32, 32)
    x2, mom2 = _layer(x1, w1t, b1t, 32)

    wp2, bp2 = _affine(w2, b2, g2, be2, mom2, 32, 32)
    w2t, b2t = _pack(wp2, bp2, 32, 64)
    y = _final(x2, w2t, b2t)

    new_points = jnp.transpose(y, (0, 2, 1))
    return (new_xyz, new_points)
